# Initial kernel scaffold; baseline (speedup 1.0000x reference)
import jax
import jax.numpy as jnp
from jax.experimental import pallas as pl


def kernel(x, edge_index, edge_attr, w_s, w_n):
    raise NotImplementedError("write your pallas kernel here")


# speedup vs baseline: 3.8207x; 3.8207x over previous
import functools

import jax
import jax.numpy as jnp
from jax import lax
from jax.experimental import pallas as pl
from jax.experimental.pallas import tpu as pltpu
from jax.experimental.pallas import tpu_sc as plsc

_NC = 2
_NS = 16
_NW = _NC * _NS
_CHUNK = 128
_PADROWS = 16


def _sc_segment_sums(dst_r, src_r, ea_r, x):
    nchunk = dst_r.shape[0]
    cp = nchunk // _NW
    n, f = x.shape
    bond = ea_r.shape[2]
    n_acc = -(-(n + _PADROWS) // 128) * 128
    zrows = n_acc // _NS

    mesh = plsc.VectorSubcoreMesh(core_axis_name="c", subcore_axis_name="s")

    @functools.partial(
        pl.kernel,
        mesh=mesh,
        compiler_params=pltpu.CompilerParams(use_tc_tiling_on_sc=False),
        out_type=[
            jax.ShapeDtypeStruct((_NC, n_acc, f), jnp.float32),
            jax.ShapeDtypeStruct((_NC, n_acc, bond), jnp.float32),
            jax.ShapeDtypeStruct((_NC, n_acc, bond), jnp.float32),
        ],
        scratch_types=[
            pltpu.VMEM((_CHUNK,), jnp.int32),
            pltpu.VMEM((_CHUNK,), jnp.int32),
            pltpu.VMEM((_CHUNK, f), jnp.float32),
            pltpu.VMEM((_CHUNK, bond), jnp.float32),
            pltpu.VMEM((_CHUNK, bond), jnp.float32),
            pltpu.VMEM_SHARED((n_acc, f), jnp.float32),
            pltpu.VMEM_SHARED((n_acc, bond), jnp.float32),
            pltpu.VMEM_SHARED((n_acc, bond), jnp.float32),
            pltpu.SemaphoreType.DMA,
        ],
    )
    def seg_kernel(dst_h, src_h, ea_h, x_h, s_out, a_out, d_out,
                   dst_v, src_v, rows_v, ea_v, ones_v,
                   s_sh, a_sh, d_sh, sem):
        cid = lax.axis_index("c")
        sid = lax.axis_index("s")
        wid = cid * _NS + sid

        zvec = jnp.zeros((16,), jnp.float32)
        ovec = jnp.ones((16,), jnp.float32)
        fv = f // 16

        def fill_rows(i, carry):
            rows_v[i // fv, pl.ds((i % fv) * 16, 16)] = zvec
            return carry

        lax.fori_loop(0, _CHUNK * fv, fill_rows, 0)

        def fill_ea(i, carry):
            ea_v[i, :] = zvec
            return carry

        lax.fori_loop(0, _CHUNK, fill_ea, 0)

        def fill_ones(i, carry):
            ones_v[i, :] = ovec
            return carry

        lax.fori_loop(0, _CHUNK, fill_ones, 0)

        zbase = sid * zrows
        nfull, rem = divmod(zrows, _CHUNK)
        for j in range(nfull):
            off = zbase + j * _CHUNK
            pltpu.sync_copy(rows_v, s_sh.at[pl.ds(off, _CHUNK)])
            pltpu.sync_copy(ea_v, a_sh.at[pl.ds(off, _CHUNK)])
            pltpu.sync_copy(ea_v, d_sh.at[pl.ds(off, _CHUNK)])
        if rem:
            off = zbase + nfull * _CHUNK
            pltpu.sync_copy(rows_v.at[pl.ds(0, rem)], s_sh.at[pl.ds(off, rem)])
            pltpu.sync_copy(ea_v.at[pl.ds(0, rem)], a_sh.at[pl.ds(off, rem)])
            pltpu.sync_copy(ea_v.at[pl.ds(0, rem)], d_sh.at[pl.ds(off, rem)])
        plsc.subcore_barrier()

        def step(c, carry):
            ch = wid * cp + c
            pltpu.sync_copy(dst_h.at[ch], dst_v)
            pltpu.sync_copy(src_h.at[ch], src_v)
            pltpu.async_copy(x_h.at[src_v], rows_v, sem).wait()
            pltpu.sync_copy(ea_h.at[ch], ea_v)
            pltpu.sync_copy(rows_v, s_sh.at[dst_v], add=True)
            pltpu.sync_copy(ea_v, a_sh.at[dst_v], add=True)
            pltpu.sync_copy(ones_v, d_sh.at[dst_v], add=True)
            return carry

        lax.fori_loop(0, cp, step, 0)
        plsc.subcore_barrier()

        pltpu.sync_copy(s_sh.at[pl.ds(zbase, zrows)],
                        s_out.at[cid, pl.ds(zbase, zrows)])
        pltpu.sync_copy(a_sh.at[pl.ds(zbase, zrows)],
                        a_out.at[cid, pl.ds(zbase, zrows)])
        pltpu.sync_copy(d_sh.at[pl.ds(zbase, zrows)],
                        d_out.at[cid, pl.ds(zbase, zrows)])

    return seg_kernel(dst_r, src_r, ea_r, x)


def _combine_body(x_ref, s_ref, a_ref, d_ref, ws_ref, wf_ref, wb_ref, o_ref):
    xb = x_ref[...]
    s = s_ref[0] + s_ref[1]
    a = a_ref[0] + a_ref[1]
    deg = d_ref[0, :, 0:1] + d_ref[1, :, 0:1]
    h = xb * deg + s
    acc = jnp.dot(xb, ws_ref[...], preferred_element_type=jnp.float32)
    acc += jnp.dot(h, wf_ref[...], preferred_element_type=jnp.float32)
    acc += jnp.dot(a, wb_ref[...], preferred_element_type=jnp.float32)
    o_ref[...] = acc


def _tc_combine(x, s, a, d, w_s, w_nf, w_nb):
    n, f = x.shape
    bond = a.shape[2]
    c_out = w_s.shape[1]
    bm = 1000
    grid = (n // bm,)
    return pl.pallas_call(
        _combine_body,
        grid=grid,
        in_specs=[
            pl.BlockSpec((bm, f), lambda i: (i, 0)),
            pl.BlockSpec((_NC, bm, f), lambda i: (0, i, 0)),
            pl.BlockSpec((_NC, bm, bond), lambda i: (0, i, 0)),
            pl.BlockSpec((_NC, bm, bond), lambda i: (0, i, 0)),
            pl.BlockSpec((f, c_out), lambda i: (0, 0)),
            pl.BlockSpec((f, c_out), lambda i: (0, 0)),
            pl.BlockSpec((bond, c_out), lambda i: (0, 0)),
        ],
        out_specs=pl.BlockSpec((bm, c_out), lambda i: (i, 0)),
        out_shape=jax.ShapeDtypeStruct((n, c_out), jnp.float32),
    )(x, s, a, d, w_s, w_nf, w_nb)


def kernel(x, edge_index, edge_attr, w_s, w_n):
    n, f = x.shape
    e = edge_index.shape[1]
    bond = edge_attr.shape[1]

    slab = _CHUNK * _NW
    e_pad = -(-e // slab) * slab
    pad = e_pad - e
    dst = edge_index[0]
    src = edge_index[1]
    if pad:
        fill = n + (jnp.arange(pad, dtype=jnp.int32) % _PADROWS)
        dst = jnp.concatenate([dst, fill])
        src = jnp.concatenate([src, jnp.zeros((pad,), jnp.int32)])
        edge_attr = jnp.concatenate(
            [edge_attr, jnp.zeros((pad, bond), edge_attr.dtype)])

    dst_r = dst.reshape(-1, _CHUNK)
    src_r = src.reshape(-1, _CHUNK)
    ea_r = edge_attr.reshape(-1, _CHUNK, bond)

    s, a, d = _sc_segment_sums(dst_r, src_r, ea_r, x)
    return _tc_combine(x, s, a, d, w_s, w_n[:f], w_n[f:])


# speedup vs baseline: 8.9458x; 2.3414x over previous
import functools

import jax
import jax.numpy as jnp
from jax import lax
from jax.experimental import pallas as pl
from jax.experimental.pallas import tpu as pltpu
from jax.experimental.pallas import tpu_sc as plsc

_NC = 2
_NS = 16
_NW = _NC * _NS
_K = 80
_G = 8
_PADROWS = 16


def _sc_segment_sums(dst_r, src_r, ea_r, x):
    nchunk = dst_r.shape[0]
    cp = nchunk // _NW
    n, f = x.shape
    bond = ea_r.shape[2]
    n_acc = -(-(n + _PADROWS) // 128) * 128
    zrows = n_acc // _NS
    ngroups = -(-cp // _G)
    lb = [min(g * _G, cp - _G) for g in range(ngroups)]

    mesh = plsc.VectorSubcoreMesh(core_axis_name="c", subcore_axis_name="s")

    @functools.partial(
        pl.kernel,
        mesh=mesh,
        compiler_params=pltpu.CompilerParams(use_tc_tiling_on_sc=False),
        out_type=[
            jax.ShapeDtypeStruct((_NC, n_acc, f), jnp.float32),
            jax.ShapeDtypeStruct((_NC, n_acc, bond), jnp.float32),
            jax.ShapeDtypeStruct((_NC, n_acc, bond), jnp.float32),
        ],
        scratch_types=[
            pltpu.VMEM((_G, _K), jnp.int32),
            pltpu.VMEM((_G, _K), jnp.int32),
            pltpu.VMEM((_G, _K), jnp.int32),
            pltpu.VMEM((_G, _K), jnp.int32),
            pltpu.VMEM((_K, f), jnp.float32),
            pltpu.VMEM((_K, f), jnp.float32),
            pltpu.VMEM((_K, bond), jnp.float32),
            pltpu.VMEM((_K, bond), jnp.float32),
            pltpu.VMEM((_K, bond), jnp.float32),
            pltpu.VMEM_SHARED((n_acc, f), jnp.float32),
            pltpu.VMEM_SHARED((n_acc, bond), jnp.float32),
            pltpu.VMEM_SHARED((n_acc, bond), jnp.float32),
            pltpu.SemaphoreType.DMA,
            pltpu.SemaphoreType.DMA,
            pltpu.SemaphoreType.DMA,
            pltpu.SemaphoreType.DMA,
            pltpu.SemaphoreType.DMA,
        ],
    )
    def seg_kernel(dst_h, src_h, ea_h, x_h, s_out, a_out, d_out,
                   di0, di1, si0, si1, rows0, rows1, eab0, eab1, ones_v,
                   s_sh, a_sh, d_sh, isem, gsem0, gsem1, ssem0, ssem1):
        cid = lax.axis_index("c")
        sid = lax.axis_index("s")
        wid = cid * _NS + sid
        base = wid * cp

        di = [di0, di1]
        si = [si0, si1]
        rows = [rows0, rows1]
        eab = [eab0, eab1]
        gsem = [gsem0, gsem1]
        ssem = [ssem0, ssem1]

        zvec = jnp.zeros((16,), jnp.float32)
        ovec = jnp.ones((16,), jnp.float32)
        fv = f // 16

        def fill_rows(i, carry):
            rows0[i // fv, pl.ds((i % fv) * 16, 16)] = zvec
            return carry

        lax.fori_loop(0, _K * fv, fill_rows, 0)

        def fill_ea(i, carry):
            eab0[i, :] = zvec
            return carry

        lax.fori_loop(0, _K, fill_ea, 0)

        def fill_ones(i, carry):
            ones_v[i, :] = ovec
            return carry

        lax.fori_loop(0, _K, fill_ones, 0)

        zbase = sid * zrows
        nfull, rem = divmod(zrows, _K)
        for j in range(nfull):
            off = zbase + j * _K
            pltpu.sync_copy(rows0, s_sh.at[pl.ds(off, _K)])
            pltpu.sync_copy(eab0, a_sh.at[pl.ds(off, _K)])
            pltpu.sync_copy(eab0, d_sh.at[pl.ds(off, _K)])
        if rem:
            off = zbase + nfull * _K
            pltpu.sync_copy(rows0.at[pl.ds(0, rem)], s_sh.at[pl.ds(off, rem)])
            pltpu.sync_copy(eab0.at[pl.ds(0, rem)], a_sh.at[pl.ds(off, rem)])
            pltpu.sync_copy(eab0.at[pl.ds(0, rem)], d_sh.at[pl.ds(off, rem)])
        plsc.subcore_barrier()

        def load_group(g):
            ib = g % 2
            sl = pl.ds(base + lb[g], _G)
            return [pltpu.async_copy(dst_h.at[sl], di[ib], isem),
                    pltpu.async_copy(src_h.at[sl], si[ib], isem)]

        idesc = [None, None]
        idesc[0] = load_group(0)
        for d in idesc[0]:
            d.wait()
        gdesc = [None, None]
        sdesc = [None, None]
        for c in range(cp + 1):
            if c < cp:
                b = c % 2
                g = c // _G
                if c % _G == 0 and c > 0:
                    for d in idesc[g % 2]:
                        d.wait()
                if sdesc[b] is not None:
                    for d in sdesc[b]:
                        d.wait()
                    sdesc[b] = None
                if c % _G == 1 and (g + 1) * _G < cp:
                    idesc[(g + 1) % 2] = load_group(g + 1)
                r = c - lb[g]
                ib = g % 2
                gdesc[b] = (
                    [pltpu.async_copy(x_h.at[si[ib].at[r]], rows[b], gsem[b]),
                     pltpu.async_copy(ea_h.at[base + c], eab[b], gsem[b])],
                    ib, r)
            if c >= 1:
                pb = (c - 1) % 2
                gds, ib, r = gdesc[pb]
                for d in gds:
                    d.wait()
                dsl = di[ib].at[r]
                sdesc[pb] = [
                    pltpu.async_copy(rows[pb], s_sh.at[dsl], ssem[pb],
                                     add=True),
                    pltpu.async_copy(eab[pb], a_sh.at[dsl], ssem[pb],
                                     add=True),
                    pltpu.async_copy(ones_v, d_sh.at[dsl], ssem[pb],
                                     add=True),
                ]
        for bb in range(2):
            if sdesc[bb] is not None:
                for d in sdesc[bb]:
                    d.wait()
        plsc.subcore_barrier()

        pltpu.sync_copy(s_sh.at[pl.ds(zbase, zrows)],
                        s_out.at[cid, pl.ds(zbase, zrows)])
        pltpu.sync_copy(a_sh.at[pl.ds(zbase, zrows)],
                        a_out.at[cid, pl.ds(zbase, zrows)])
        pltpu.sync_copy(d_sh.at[pl.ds(zbase, zrows)],
                        d_out.at[cid, pl.ds(zbase, zrows)])

    return seg_kernel(dst_r, src_r, ea_r, x)


def _combine_body(x_ref, s_ref, a_ref, d_ref, ws_ref, wf_ref, wb_ref, o_ref):
    xb = x_ref[...]
    s = s_ref[0] + s_ref[1]
    a = a_ref[0] + a_ref[1]
    deg = d_ref[0, :, 0:1] + d_ref[1, :, 0:1]
    h = xb * deg + s
    acc = jnp.dot(xb, ws_ref[...], preferred_element_type=jnp.float32)
    acc += jnp.dot(h, wf_ref[...], preferred_element_type=jnp.float32)
    acc += jnp.dot(a, wb_ref[...], preferred_element_type=jnp.float32)
    o_ref[...] = acc


def _tc_combine(x, s, a, d, w_s, w_nf, w_nb):
    n, f = x.shape
    bond = a.shape[2]
    c_out = w_s.shape[1]
    bm = 1000
    grid = (n // bm,)
    return pl.pallas_call(
        _combine_body,
        grid=grid,
        in_specs=[
            pl.BlockSpec((bm, f), lambda i: (i, 0)),
            pl.BlockSpec((_NC, bm, f), lambda i: (0, i, 0)),
            pl.BlockSpec((_NC, bm, bond), lambda i: (0, i, 0)),
            pl.BlockSpec((_NC, bm, bond), lambda i: (0, i, 0)),
            pl.BlockSpec((f, c_out), lambda i: (0, 0)),
            pl.BlockSpec((f, c_out), lambda i: (0, 0)),
            pl.BlockSpec((bond, c_out), lambda i: (0, 0)),
        ],
        out_specs=pl.BlockSpec((bm, c_out), lambda i: (i, 0)),
        out_shape=jax.ShapeDtypeStruct((n, c_out), jnp.float32),
    )(x, s, a, d, w_s, w_nf, w_nb)


def kernel(x, edge_index, edge_attr, w_s, w_n):
    n, f = x.shape
    e = edge_index.shape[1]
    bond = edge_attr.shape[1]

    slab = _K * _NW
    e_pad = -(-e // slab) * slab
    pad = e_pad - e
    dst = edge_index[0]
    src = edge_index[1]
    if pad:
        fill = n + (jnp.arange(pad, dtype=jnp.int32) % _PADROWS)
        dst = jnp.concatenate([dst, fill])
        src = jnp.concatenate([src, jnp.zeros((pad,), jnp.int32)])
        edge_attr = jnp.concatenate(
            [edge_attr, jnp.zeros((pad, bond), edge_attr.dtype)])

    dst_r = dst.reshape(-1, _K)
    src_r = src.reshape(-1, _K)
    ea_r = edge_attr.reshape(-1, _K, bond)

    s, a, d = _sc_segment_sums(dst_r, src_r, ea_r, x)
    return _tc_combine(x, s, a, d, w_s, w_n[:f], w_n[f:])
